# baseline (speedup 1.0000x reference)
import functools

import jax
import jax.numpy as jnp
from jax import lax
from jax.experimental import pallas as pl
from jax.experimental.pallas import tpu as pltpu
from jax.experimental.pallas import tpu_sc as plsc

N = 10000
E = 320000
D = 128

NC = 2
NS = 16
L = 16
NW = NC * NS

NP = 10240
ROWS = E // 128
TROWS = 80
LAST_ROWS = 16
TAIL_ROWS = ROWS - (NW - 1) * TROWS - LAST_ROWS
SLICE = NP // NS


def _proj_body(feat_ref, wl_ref, bl_ref, wr_ref, br_ref, fsrc_ref, fdst_ref):
    x = feat_ref[...]
    dn = (((1,), (1,)), ((), ()))
    fsrc_ref[...] = (
        lax.dot_general(wl_ref[...], x, dn,
                        preferred_element_type=jnp.float32) + bl_ref[...])
    fdst_ref[...] = (
        lax.dot_general(wr_ref[...], x, dn,
                        preferred_element_type=jnp.float32) + br_ref[...])


_proj_call = pl.pallas_call(
    _proj_body,
    out_shape=(
        jax.ShapeDtypeStruct((1, N), jnp.float32),
        jax.ShapeDtypeStruct((1, N), jnp.float32),
    ),
)


@functools.cache
def _make_edge_kernel():
    mesh = plsc.VectorSubcoreMesh(core_axis_name="c", subcore_axis_name="s")

    @functools.partial(
        pl.kernel,
        out_type=(
            jax.ShapeDtypeStruct((1, NC * NP), jnp.float32),
            jax.ShapeDtypeStruct((1, NC * NP), jnp.float32),
        ),
        mesh=mesh,
        scratch_types=[
            pltpu.VMEM((2, TROWS * 128), jnp.int32),
            pltpu.VMEM((TROWS * 128,), jnp.int32),
            pltpu.VMEM((TROWS * 128,), jnp.float32),
            pltpu.VMEM((N,), jnp.float32),
            pltpu.VMEM((SLICE,), jnp.float32),
            pltpu.VMEM((TROWS * 128,), jnp.float32),
            pltpu.VMEM((2, 128), jnp.int32),
            pltpu.VMEM((8, 128), jnp.float32),
            pltpu.VMEM_SHARED((NP,), jnp.float32),
            pltpu.VMEM_SHARED((NP,), jnp.float32),
            pltpu.SemaphoreType.DMA,
        ],
        compiler_params=pltpu.CompilerParams(needs_layout_passes=False),
    )
    def _edge_kernel(fdst_hbm, edge_hbm, seg_out, deg_out,
                     ev, didx_v, vals_v, table_v, zeros_v, ones_v,
                     tev, tvals_v,
                     seg_sh, deg_sh, ssem):
        cid = lax.axis_index("c")
        sid = lax.axis_index("s")
        wid = sid * NC + cid
        col0 = wid * TROWS * 128

        @pl.loop(0, SLICE // L)
        def _(i):
            zeros_v[pl.ds(i * L, L)] = jnp.zeros((L,), jnp.float32)

        pltpu.async_copy(fdst_hbm.at[0], table_v, ssem)

        @pl.when(wid < NW - 1)
        def _():
            pltpu.async_copy(edge_hbm.at[:, pl.ds(col0, TROWS * 128)], ev,
                             ssem)

        @pl.when(wid == NW - 1)
        def _():
            pltpu.async_copy(edge_hbm.at[:, pl.ds(col0, LAST_ROWS * 128)],
                             ev.at[:, pl.ds(0, LAST_ROWS * 128)], ssem)

        @pl.when(wid >= NW - TAIL_ROWS)
        def _():
            base = ((NW - 1) * TROWS + LAST_ROWS + wid - (NW - TAIL_ROWS))
            pltpu.async_copy(edge_hbm.at[:, pl.ds(base * 128, 128)], tev,
                             ssem)

        pltpu.sync_copy(zeros_v, seg_sh.at[pl.ds(sid * SLICE, SLICE)])
        pltpu.sync_copy(zeros_v, deg_sh.at[pl.ds(sid * SLICE, SLICE)])

        pltpu.make_async_copy(fdst_hbm.at[0], table_v, ssem).wait()

        @pl.when(wid < NW - 1)
        def _():
            pltpu.make_async_copy(
                edge_hbm.at[:, pl.ds(col0, TROWS * 128)], ev, ssem).wait()

        @pl.when(wid == NW - 1)
        def _():
            pltpu.make_async_copy(
                edge_hbm.at[:, pl.ds(col0, LAST_ROWS * 128)],
                ev.at[:, pl.ds(0, LAST_ROWS * 128)], ssem).wait()

        @pl.when(wid >= NW - TAIL_ROWS)
        def _():
            base = ((NW - 1) * TROWS + LAST_ROWS + wid - (NW - TAIL_ROWS))
            pltpu.make_async_copy(edge_hbm.at[:, pl.ds(base * 128, 128)],
                                  tev, ssem).wait()

        plsc.subcore_barrier()

        def gather_16(p):
            idx = ev[0, pl.ds(p, L)]
            vals_v[pl.ds(p, L)] = plsc.load_gather(table_v, [idx])
            didx_v[pl.ds(p, L)] = ev[1, pl.ds(p, L)]
            ones_v[pl.ds(p, L)] = jnp.ones((L,), jnp.float32)

        @pl.loop(0, LAST_ROWS * 8)
        def _(g):
            gather_16(g * L)

        @pl.when(wid < NW - 1)
        def _():
            @pl.loop(LAST_ROWS * 8, TROWS * 8)
            def _(g):
                gather_16(g * L)

        def fire_and_drain(nrows):
            didx = didx_v.at[pl.ds(0, nrows * 128)]
            vsl = vals_v.at[pl.ds(0, nrows * 128)]
            osl = ones_v.at[pl.ds(0, nrows * 128)]
            pltpu.async_copy(vsl, seg_sh.at[didx], ssem, add=True)
            pltpu.async_copy(osl, deg_sh.at[didx], ssem, add=True)
            pltpu.make_async_copy(vsl, seg_sh.at[didx], ssem).wait()
            pltpu.make_async_copy(osl, deg_sh.at[didx], ssem).wait()

        @pl.when(wid < NW - 1)
        def _():
            fire_and_drain(TROWS)

        @pl.when(wid == NW - 1)
        def _():
            fire_and_drain(LAST_ROWS)

        @pl.when(wid >= NW - TAIL_ROWS)
        def _():
            for k in range(128 // L):
                idx = tev[0, pl.ds(k * L, L)]
                tvals_v[0, pl.ds(k * L, L)] = plsc.load_gather(table_v, [idx])
            tidx = tev.at[1]
            tosl = ones_v.at[pl.ds(0, 128)]
            pltpu.async_copy(tvals_v.at[0], seg_sh.at[tidx], ssem, add=True)
            pltpu.async_copy(tosl, deg_sh.at[tidx], ssem, add=True)
            pltpu.make_async_copy(tvals_v.at[0], seg_sh.at[tidx],
                                  ssem).wait()
            pltpu.make_async_copy(tosl, deg_sh.at[tidx], ssem).wait()

        plsc.subcore_barrier()

        out0 = cid * NP + sid * SLICE
        pltpu.sync_copy(seg_sh.at[pl.ds(sid * SLICE, SLICE)],
                        seg_out.at[0, pl.ds(out0, SLICE)])
        pltpu.sync_copy(deg_sh.at[pl.ds(sid * SLICE, SLICE)],
                        deg_out.at[0, pl.ds(out0, SLICE)])

    return _edge_kernel


def _final_body(fsrc_ref, seg_ref, deg_ref, out_ref):
    fsrc = fsrc_ref[0]
    seg = seg_ref[0]
    deg = deg_ref[0]
    seg_sum = seg[:N] + seg[NP:NP + N]
    deg_sum = deg[:N] + deg[NP:NP + N]
    feat_d = jax.nn.relu(seg_sum + fsrc) + 1.0
    g_u = jax.nn.relu(fsrc) + 1.0
    degree = jnp.maximum(deg_sum, 1.0)
    p1 = jnp.sqrt(degree) * feat_d * g_u
    out_ref[0, :] = p1 * (float(int(N * 0.25)) / jnp.sum(p1))


_final_call = pl.pallas_call(
    _final_body,
    out_shape=jax.ShapeDtypeStruct((1, N), jnp.float32),
)


def kernel(feat, edge_index, W_l, b_l, W_r, b_r):
    fsrc, fdst = _proj_call(feat, W_l, b_l, W_r, b_r)
    seg, deg = _make_edge_kernel()(fdst, edge_index)
    p1 = _final_call(fsrc, seg, deg)
    return p1.reshape(N, 1)


# speedup vs baseline: 1.1309x; 1.1309x over previous
import functools

import jax
import jax.numpy as jnp
from jax import lax
from jax.experimental import pallas as pl
from jax.experimental.pallas import tpu as pltpu
from jax.experimental.pallas import tpu_sc as plsc

N = 10000
E = 320000
D = 128

NC = 2
NS = 16
L = 16
NW = NC * NS

NP = 10240
ROWS = E // 128
TROWS = 80
LAST_ROWS = 16
TAIL_ROWS = ROWS - (NW - 1) * TROWS - LAST_ROWS
SLICE = NP // NS


def _proj_body(feat_ref, wl_ref, bl_ref, wr_ref, br_ref, fsrc_ref, fdst_ref):
    x = feat_ref[...]
    dn = (((1,), (1,)), ((), ()))
    fsrc_ref[...] = (
        lax.dot_general(wl_ref[...], x, dn,
                        preferred_element_type=jnp.float32) + bl_ref[...])
    fdst_ref[...] = (
        lax.dot_general(wr_ref[...], x, dn,
                        preferred_element_type=jnp.float32) + br_ref[...])


_proj_call = pl.pallas_call(
    _proj_body,
    out_shape=(
        jax.ShapeDtypeStruct((1, N), jnp.float32),
        jax.ShapeDtypeStruct((1, N), jnp.float32),
    ),
)


@functools.cache
def _make_edge_kernel():
    mesh = plsc.VectorSubcoreMesh(core_axis_name="c", subcore_axis_name="s")

    @functools.partial(
        pl.kernel,
        out_type=(
            jax.ShapeDtypeStruct((1, NC * NP), jnp.float32),
            jax.ShapeDtypeStruct((1, NC * NP), jnp.float32),
        ),
        mesh=mesh,
        scratch_types=[
            pltpu.VMEM((2, TROWS * 128), jnp.int32),
            pltpu.VMEM((TROWS * 128,), jnp.int32),
            pltpu.VMEM((TROWS * 128,), jnp.float32),
            pltpu.VMEM((N,), jnp.float32),
            pltpu.VMEM((SLICE,), jnp.float32),
            pltpu.VMEM((TROWS * 128,), jnp.float32),
            pltpu.VMEM((2, 128), jnp.int32),
            pltpu.VMEM((8, 128), jnp.float32),
            pltpu.VMEM_SHARED((NP,), jnp.float32),
            pltpu.VMEM_SHARED((NP,), jnp.float32),
            pltpu.SemaphoreType.DMA,
        ],
        compiler_params=pltpu.CompilerParams(needs_layout_passes=False),
    )
    def _edge_kernel(fdst_hbm, edge_hbm, seg_out, deg_out,
                     ev, didx_v, vals_v, table_v, zeros_v, ones_v,
                     tev, tvals_v,
                     seg_sh, deg_sh, ssem):
        cid = lax.axis_index("c")
        sid = lax.axis_index("s")
        wid = sid * NC + cid
        col0 = wid * TROWS * 128

        @pl.loop(0, SLICE // L)
        def _(i):
            zeros_v[pl.ds(i * L, L)] = jnp.zeros((L,), jnp.float32)

        pltpu.async_copy(fdst_hbm.at[0], table_v, ssem)

        @pl.when(wid < NW - 1)
        def _():
            pltpu.async_copy(edge_hbm.at[:, pl.ds(col0, TROWS * 128)], ev,
                             ssem)

        @pl.when(wid == NW - 1)
        def _():
            pltpu.async_copy(edge_hbm.at[:, pl.ds(col0, LAST_ROWS * 128)],
                             ev.at[:, pl.ds(0, LAST_ROWS * 128)], ssem)

        @pl.when(wid >= NW - TAIL_ROWS)
        def _():
            base = ((NW - 1) * TROWS + LAST_ROWS + wid - (NW - TAIL_ROWS))
            pltpu.async_copy(edge_hbm.at[:, pl.ds(base * 128, 128)], tev,
                             ssem)

        pltpu.sync_copy(zeros_v, seg_sh.at[pl.ds(sid * SLICE, SLICE)])
        pltpu.sync_copy(zeros_v, deg_sh.at[pl.ds(sid * SLICE, SLICE)])

        pltpu.make_async_copy(fdst_hbm.at[0], table_v, ssem).wait()

        @pl.when(wid < NW - 1)
        def _():
            pltpu.make_async_copy(
                edge_hbm.at[:, pl.ds(col0, TROWS * 128)], ev, ssem).wait()

        @pl.when(wid == NW - 1)
        def _():
            pltpu.make_async_copy(
                edge_hbm.at[:, pl.ds(col0, LAST_ROWS * 128)],
                ev.at[:, pl.ds(0, LAST_ROWS * 128)], ssem).wait()

        @pl.when(wid >= NW - TAIL_ROWS)
        def _():
            base = ((NW - 1) * TROWS + LAST_ROWS + wid - (NW - TAIL_ROWS))
            pltpu.make_async_copy(edge_hbm.at[:, pl.ds(base * 128, 128)],
                                  tev, ssem).wait()

        plsc.subcore_barrier()

        CH = 512

        def gather_16(p):
            idx = ev[0, pl.ds(p, L)]
            vals_v[pl.ds(p, L)] = plsc.load_gather(table_v, [idx])
            didx_v[pl.ds(p, L)] = ev[1, pl.ds(p, L)]
            ones_v[pl.ds(p, L)] = jnp.ones((L,), jnp.float32)

        def do_chunk(c):
            @pl.loop(0, CH // L)
            def _(g):
                gather_16(c * CH + g * L)
            sl = pl.ds(c * CH, CH)
            pltpu.async_copy(vals_v.at[sl], seg_sh.at[didx_v.at[sl]], ssem,
                             add=True)
            pltpu.async_copy(ones_v.at[sl], deg_sh.at[didx_v.at[sl]], ssem,
                             add=True)

        @pl.loop(0, LAST_ROWS * 128 // CH)
        def _(c):
            do_chunk(c)

        @pl.when(wid < NW - 1)
        def _():
            @pl.loop(LAST_ROWS * 128 // CH, TROWS * 128 // CH)
            def _(c):
                do_chunk(c)

        def drain(nrows):
            didx = didx_v.at[pl.ds(0, nrows * 128)]
            vsl = vals_v.at[pl.ds(0, nrows * 128)]
            osl = ones_v.at[pl.ds(0, nrows * 128)]
            pltpu.make_async_copy(vsl, seg_sh.at[didx], ssem).wait()
            pltpu.make_async_copy(osl, deg_sh.at[didx], ssem).wait()

        @pl.when(wid < NW - 1)
        def _():
            drain(TROWS)

        @pl.when(wid == NW - 1)
        def _():
            drain(LAST_ROWS)

        @pl.when(wid >= NW - TAIL_ROWS)
        def _():
            for k in range(128 // L):
                idx = tev[0, pl.ds(k * L, L)]
                tvals_v[0, pl.ds(k * L, L)] = plsc.load_gather(table_v, [idx])
            tidx = tev.at[1]
            tosl = ones_v.at[pl.ds(0, 128)]
            pltpu.async_copy(tvals_v.at[0], seg_sh.at[tidx], ssem, add=True)
            pltpu.async_copy(tosl, deg_sh.at[tidx], ssem, add=True)
            pltpu.make_async_copy(tvals_v.at[0], seg_sh.at[tidx],
                                  ssem).wait()
            pltpu.make_async_copy(tosl, deg_sh.at[tidx], ssem).wait()

        plsc.subcore_barrier()

        out0 = cid * NP + sid * SLICE
        pltpu.sync_copy(seg_sh.at[pl.ds(sid * SLICE, SLICE)],
                        seg_out.at[0, pl.ds(out0, SLICE)])
        pltpu.sync_copy(deg_sh.at[pl.ds(sid * SLICE, SLICE)],
                        deg_out.at[0, pl.ds(out0, SLICE)])

    return _edge_kernel


def _final_body(fsrc_ref, seg_ref, deg_ref, out_ref):
    fsrc = fsrc_ref[0]
    seg = seg_ref[0]
    deg = deg_ref[0]
    seg_sum = seg[:N] + seg[NP:NP + N]
    deg_sum = deg[:N] + deg[NP:NP + N]
    feat_d = jax.nn.relu(seg_sum + fsrc) + 1.0
    g_u = jax.nn.relu(fsrc) + 1.0
    degree = jnp.maximum(deg_sum, 1.0)
    p1 = jnp.sqrt(degree) * feat_d * g_u
    out_ref[0, :] = p1 * (float(int(N * 0.25)) / jnp.sum(p1))


_final_call = pl.pallas_call(
    _final_body,
    out_shape=jax.ShapeDtypeStruct((1, N), jnp.float32),
)


def kernel(feat, edge_index, W_l, b_l, W_r, b_r):
    fsrc, fdst = _proj_call(feat, W_l, b_l, W_r, b_r)
    seg, deg = _make_edge_kernel()(fdst, edge_index)
    p1 = _final_call(fsrc, seg, deg)
    return p1.reshape(N, 1)


# speedup vs baseline: 1.2376x; 1.0944x over previous
import functools

import jax
import jax.numpy as jnp
from jax import lax
from jax.experimental import pallas as pl
from jax.experimental.pallas import tpu as pltpu
from jax.experimental.pallas import tpu_sc as plsc

N = 10000
E = 320000
D = 128

NC = 2
NS = 16
L = 16
NW = NC * NS

NP = 10240
ROWS = E // 128
TROWS = 80
LAST_ROWS = 16
TAIL_ROWS = ROWS - (NW - 1) * TROWS - LAST_ROWS
SLICE = NP // NS


def _matvec_body(feat_ref, w_ref, b_ref, out_ref):
    dn = (((1,), (1,)), ((), ()))
    out_ref[...] = (
        lax.dot_general(w_ref[...], feat_ref[...], dn,
                        preferred_element_type=jnp.float32) + b_ref[...])


_matvec_call = pl.pallas_call(
    _matvec_body,
    out_shape=jax.ShapeDtypeStruct((1, N), jnp.float32),
)


@functools.cache
def _make_edge_kernel():
    mesh = plsc.VectorSubcoreMesh(core_axis_name="c", subcore_axis_name="s")

    @functools.partial(
        pl.kernel,
        out_type=(
            jax.ShapeDtypeStruct((1, NC * NP), jnp.float32),
            jax.ShapeDtypeStruct((1, NC * NP), jnp.float32),
        ),
        mesh=mesh,
        scratch_types=[
            pltpu.VMEM((2, TROWS * 128), jnp.int32),
            pltpu.VMEM((TROWS * 128,), jnp.float32),
            pltpu.VMEM((N,), jnp.float32),
            pltpu.VMEM((SLICE,), jnp.float32),
            pltpu.VMEM((128,), jnp.float32),
            pltpu.VMEM((2, 128), jnp.int32),
            pltpu.VMEM((8, 128), jnp.float32),
            pltpu.VMEM_SHARED((NP,), jnp.float32),
            pltpu.VMEM_SHARED((NP,), jnp.float32),
            pltpu.SemaphoreType.DMA,
        ],
        compiler_params=pltpu.CompilerParams(needs_layout_passes=False),
    )
    def _edge_kernel(fdst_hbm, edge_hbm, seg_out, deg_out,
                     ev, vals_v, table_v, zeros_v, ones_v,
                     tev, tvals_v,
                     seg_sh, deg_sh, ssem):
        cid = lax.axis_index("c")
        sid = lax.axis_index("s")
        wid = sid * NC + cid
        col0 = wid * TROWS * 128

        @pl.loop(0, SLICE // L)
        def _(i):
            zeros_v[pl.ds(i * L, L)] = jnp.zeros((L,), jnp.float32)

        for i in range(128 // L):
            ones_v[pl.ds(i * L, L)] = jnp.ones((L,), jnp.float32)

        pltpu.async_copy(fdst_hbm.at[0], table_v, ssem)

        @pl.when(wid < NW - 1)
        def _():
            pltpu.async_copy(edge_hbm.at[:, pl.ds(col0, TROWS * 128)], ev,
                             ssem)

        @pl.when(wid == NW - 1)
        def _():
            pltpu.async_copy(edge_hbm.at[:, pl.ds(col0, LAST_ROWS * 128)],
                             ev.at[:, pl.ds(0, LAST_ROWS * 128)], ssem)

        @pl.when(wid >= NW - TAIL_ROWS)
        def _():
            base = ((NW - 1) * TROWS + LAST_ROWS + wid - (NW - TAIL_ROWS))
            pltpu.async_copy(edge_hbm.at[:, pl.ds(base * 128, 128)], tev,
                             ssem)

        pltpu.sync_copy(zeros_v, seg_sh.at[pl.ds(sid * SLICE, SLICE)])
        pltpu.sync_copy(zeros_v, deg_sh.at[pl.ds(sid * SLICE, SLICE)])

        pltpu.make_async_copy(fdst_hbm.at[0], table_v, ssem).wait()

        @pl.when(wid < NW - 1)
        def _():
            pltpu.make_async_copy(
                edge_hbm.at[:, pl.ds(col0, TROWS * 128)], ev, ssem).wait()

        @pl.when(wid == NW - 1)
        def _():
            pltpu.make_async_copy(
                edge_hbm.at[:, pl.ds(col0, LAST_ROWS * 128)],
                ev.at[:, pl.ds(0, LAST_ROWS * 128)], ssem).wait()

        @pl.when(wid >= NW - TAIL_ROWS)
        def _():
            base = ((NW - 1) * TROWS + LAST_ROWS + wid - (NW - TAIL_ROWS))
            pltpu.make_async_copy(edge_hbm.at[:, pl.ds(base * 128, 128)],
                                  tev, ssem).wait()

        plsc.subcore_barrier()

        def do_row(j):
            for k in range(128 // L):
                p = j * 128 + k * L
                idx = ev[0, pl.ds(p, L)]
                vals_v[pl.ds(p, L)] = plsc.load_gather(table_v, [idx])
            didx = ev.at[1, pl.ds(j * 128, 128)]
            pltpu.async_copy(vals_v.at[pl.ds(j * 128, 128)],
                             seg_sh.at[didx], ssem, add=True)
            pltpu.async_copy(ones_v, deg_sh.at[didx], ssem, add=True)

        @pl.loop(0, LAST_ROWS)
        def _(j):
            do_row(j)

        @pl.when(wid < NW - 1)
        def _():
            @pl.loop(LAST_ROWS, TROWS)
            def _(j):
                do_row(j)

        @pl.when(wid >= NW - TAIL_ROWS)
        def _():
            for k in range(128 // L):
                idx = tev[0, pl.ds(k * L, L)]
                tvals_v[0, pl.ds(k * L, L)] = plsc.load_gather(table_v, [idx])
            tidx = tev.at[1]
            pltpu.async_copy(tvals_v.at[0], seg_sh.at[tidx], ssem, add=True)
            pltpu.async_copy(ones_v, deg_sh.at[tidx], ssem, add=True)

        @pl.when(wid < NW - 1)
        def _():
            for _i in range(2):
                pltpu.make_async_copy(
                    seg_out.at[0, pl.ds(0, TROWS * 128)], vals_v, ssem).wait()

        @pl.when(wid == NW - 1)
        def _():
            for _i in range(2):
                pltpu.make_async_copy(
                    seg_out.at[0, pl.ds(0, LAST_ROWS * 128)],
                    vals_v.at[pl.ds(0, LAST_ROWS * 128)], ssem).wait()

        @pl.when(wid >= NW - TAIL_ROWS)
        def _():
            for _i in range(2):
                pltpu.make_async_copy(
                    seg_out.at[0, pl.ds(0, 128)], tvals_v.at[0], ssem).wait()

        plsc.subcore_barrier()

        out0 = cid * NP + sid * SLICE
        pltpu.sync_copy(seg_sh.at[pl.ds(sid * SLICE, SLICE)],
                        seg_out.at[0, pl.ds(out0, SLICE)])
        pltpu.sync_copy(deg_sh.at[pl.ds(sid * SLICE, SLICE)],
                        deg_out.at[0, pl.ds(out0, SLICE)])

    return _edge_kernel


def _final_body(fsrc_ref, seg_ref, deg_ref, out_ref):
    fsrc = fsrc_ref[0]
    seg = seg_ref[0]
    deg = deg_ref[0]
    seg_sum = seg[:N] + seg[NP:NP + N]
    deg_sum = deg[:N] + deg[NP:NP + N]
    feat_d = jax.nn.relu(seg_sum + fsrc) + 1.0
    g_u = jax.nn.relu(fsrc) + 1.0
    degree = jnp.maximum(deg_sum, 1.0)
    p1 = jnp.sqrt(degree) * feat_d * g_u
    out_ref[0, :] = p1 * (float(int(N * 0.25)) / jnp.sum(p1))


_final_call = pl.pallas_call(
    _final_body,
    out_shape=jax.ShapeDtypeStruct((1, N), jnp.float32),
)


def kernel(feat, edge_index, W_l, b_l, W_r, b_r):
    fdst = _matvec_call(feat, W_r, b_r)
    seg, deg = _make_edge_kernel()(fdst, edge_index)
    fsrc = _matvec_call(feat, W_l, b_l)
    p1 = _final_call(fsrc, seg, deg)
    return p1.reshape(N, 1)
